# baseline (speedup 1.0000x reference)
import jax
import jax.numpy as jnp
from jax import lax
from jax.experimental import pallas as pl
from jax.experimental.pallas import tpu as pltpu
from jax.experimental.pallas import tpu_sc as plsc

N_NODES = 10000
N_EDGES = 320000
D_FEAT = 128

NC = 2
NS = 16
NW = NC * NS

CH = 128
K = 13
SB = 6
CPW = K * SB
NCHT = N_EDGES // CH
TAIL0 = NW * CPW
N_TAIL = NCHT - TAIL0
TW0 = 24

NP = 10240
ZROWS = NP // NS


def _sc_body(x_hbm, e_hbm, p_hbm,
             src_a, src_b2, dst_a, dst_b2, rows_a, rows_b, acc,
             sem_a, sem_b, sem_ia, sem_ib):
    c = lax.axis_index("c")
    s = lax.axis_index("s")
    wid = s * NC + c
    wchunk0 = wid * CPW

    def idx_copies(block, srcbuf, dstbuf, sem):
        off = (wchunk0 + block * K) * CH
        cps = [pltpu.make_async_copy(e_hbm.at[0, pl.ds(off, K * CH)],
                                     srcbuf, sem)]
        for j in range(K):
            cps.append(pltpu.make_async_copy(
                e_hbm.at[1, pl.ds(off + j * CH, CH)], dstbuf.at[j], sem))
        return cps

    def idx_load(block, srcbuf, dstbuf, sem):
        for cp in idx_copies(block, srcbuf, dstbuf, sem):
            cp.start()

    def idx_wait(block, srcbuf, dstbuf, sem):
        for cp in idx_copies(block, srcbuf, dstbuf, sem):
            cp.wait()

    idx_load(0, src_a, dst_a, sem_ia)
    idx_load(1, src_b2, dst_b2, sem_ib)

    @pl.loop(0, CH)
    def _(r):
        for c0 in range(0, D_FEAT, 16):
            rows_a[r, pl.ds(c0, 16)] = jnp.zeros((16,), jnp.float32)

    @pl.loop(0, ZROWS // CH)
    def _(i):
        pltpu.sync_copy(rows_a, acc.at[pl.ds(s * ZROWS + i * CH, CH)])

    idx_wait(0, src_a, dst_a, sem_ia)
    pltpu.make_async_copy(x_hbm.at[src_a.at[pl.ds(0, CH)]],
                          rows_a, sem_a).start()
    plsc.subcore_barrier()

    rows = [rows_a, rows_b]
    sems = [sem_a, sem_b]
    srcs = [src_a, src_b2]
    dsts = [dst_a, dst_b2]
    isems = [sem_ia, sem_ib]

    @pl.loop(0, SB // 2)
    def _(q):
        for half in range(2):
            b = 2 * q + half
            scur, dcur = srcs[half], dsts[half]
            snxt, dnxt = srcs[1 - half], dsts[1 - half]
            for j in range(K):
                pg = (half + j) % 2
                png = (half + j + 1) % 2
                if j + 1 < K:
                    pltpu.make_async_copy(
                        x_hbm.at[scur.at[pl.ds((j + 1) * CH, CH)]],
                        rows[png], sems[png]).start()
                else:
                    def cross():
                        idx_wait(b + 1, snxt, dnxt, isems[1 - half])
                        pltpu.make_async_copy(
                            x_hbm.at[snxt.at[pl.ds(0, CH)]],
                            rows[png], sems[png]).start()
                    if half == 0:
                        cross()
                    else:
                        pl.when(q < SB // 2 - 1)(cross)
                pltpu.make_async_copy(x_hbm.at[scur.at[pl.ds(j * CH, CH)]],
                                      rows[pg], sems[pg]).wait()
                pltpu.sync_copy(rows[pg], acc.at[dcur.at[j]], add=True)
            @pl.when(q < SB // 2 - 1)
            def _():
                idx_load(b + 2, scur, dcur, isems[half])

    tidx = wid - TW0
    @pl.when((tidx >= 0) & (tidx < N_TAIL))
    def _():
        off = (TAIL0 + tidx) * CH
        pltpu.sync_copy(e_hbm.at[0, pl.ds(off, CH)], src_a.at[pl.ds(0, CH)])
        pltpu.sync_copy(e_hbm.at[1, pl.ds(off, CH)], dst_a.at[0])
        pltpu.sync_copy(x_hbm.at[src_a.at[pl.ds(0, CH)]], rows_a)
        pltpu.sync_copy(rows_a, acc.at[dst_a.at[0]], add=True)

    plsc.subcore_barrier()
    pltpu.sync_copy(acc.at[pl.ds(s * ZROWS, ZROWS)],
                    p_hbm.at[c, pl.ds(s * ZROWS, ZROWS)])


@jax.jit
def _sc_scatter(x, edge_index):
    mesh = plsc.VectorSubcoreMesh(core_axis_name="c", subcore_axis_name="s")
    run = pl.kernel(
        _sc_body,
        out_type=jax.ShapeDtypeStruct((NC, NP, D_FEAT), jnp.float32),
        mesh=mesh,
        scratch_types=[
            pltpu.VMEM((K * CH,), jnp.int32),
            pltpu.VMEM((K * CH,), jnp.int32),
            pltpu.VMEM((K, CH), jnp.int32),
            pltpu.VMEM((K, CH), jnp.int32),
            pltpu.VMEM((CH, D_FEAT), jnp.float32),
            pltpu.VMEM((CH, D_FEAT), jnp.float32),
            pltpu.VMEM_SHARED((NP, D_FEAT), jnp.float32),
            pltpu.SemaphoreType.DMA,
            pltpu.SemaphoreType.DMA,
            pltpu.SemaphoreType.DMA,
            pltpu.SemaphoreType.DMA,
        ],
    )
    return run(x, edge_index)


def _combine_body(p_ref, o_ref):
    o_ref[...] = p_ref[0, :N_NODES, :] + p_ref[1, :N_NODES, :]


@jax.jit
def _combine(p):
    return pl.pallas_call(
        _combine_body,
        out_shape=jax.ShapeDtypeStruct((N_NODES, D_FEAT), jnp.float32),
    )(p)


def kernel(x, edge_index):
    return _combine(_sc_scatter(x, edge_index))
